# Initial kernel scaffold; baseline (speedup 1.0000x reference)
import jax
import jax.numpy as jnp
from jax.experimental import pallas as pl


def kernel(input_ids, token_type_ids, input_mask, visual_embeds, visual_mask, tok_table, seg_table, v_seg_table, norm_gamma, norm_beta, vln_gamma, vln_beta):
    raise NotImplementedError("write your pallas kernel here")


# speedup vs baseline: 5.0762x; 5.0762x over previous
import functools

import jax
import jax.numpy as jnp
from jax import lax
from jax.experimental import pallas as pl
from jax.experimental.pallas import tpu as pltpu
from jax.experimental.pallas import tpu_sc as plsc

_EPS = 1e-12

_NC = 2
_NS = 16
_NW = _NC * _NS
_CHUNK = 128


def _sc_gather(table, ids_flat):
    n = ids_flat.shape[0]
    d = table.shape[1]
    n_chunks = n // (_NW * _CHUNK)
    idx = ids_flat.reshape(_NW, n_chunks, _CHUNK)
    mesh = plsc.VectorSubcoreMesh(core_axis_name="c", subcore_axis_name="s")

    @functools.partial(
        pl.kernel,
        mesh=mesh,
        out_type=jax.ShapeDtypeStruct((_NW, n_chunks, _CHUNK, d), jnp.float32),
        scratch_types=[
            pltpu.VMEM((n_chunks, _CHUNK), jnp.int32),
            pltpu.VMEM((_CHUNK, d), jnp.float32),
            pltpu.SemaphoreType.DMA,
        ],
    )
    def k(table_hbm, idx_hbm, out_hbm, idx_v, rows_v, sem):
        wid = lax.axis_index("s") * _NC + lax.axis_index("c")
        pltpu.sync_copy(idx_hbm.at[wid], idx_v)

        def body(c, carry):
            pltpu.async_copy(table_hbm.at[idx_v.at[c]], rows_v, sem).wait()
            pltpu.sync_copy(rows_v, out_hbm.at[wid, c])
            return carry

        lax.fori_loop(0, n_chunks, body, 0)

    return k(table, idx).reshape(n, d)


def _ln(x, g, b):
    mean = jnp.mean(x, axis=-1, keepdims=True)
    xc = x - mean
    var = jnp.mean(xc * xc, axis=-1, keepdims=True)
    return xc * lax.rsqrt(var + _EPS) * g + b


def _tc_body(tg_ref, tt_ref, vis_ref, seg_ref, vseg_ref, g_ref, b_ref,
             vg_ref, vb_ref, out_ref):
    seq_l = tg_ref.shape[1]
    ttf = tt_ref[...]
    seg0 = seg_ref[0, :]
    dseg = seg_ref[1, :] - seg0
    text = tg_ref[...] + seg0 + ttf * dseg
    g = g_ref[...]
    b = b_ref[...]
    out_ref[:, :seq_l, :] = _ln(text, g, b)
    v = _ln(vis_ref[...], vg_ref[...], vb_ref[...]) + vseg_ref[0, :]
    out_ref[:, seq_l:, :] = _ln(v, g, b)


def _tc_fuse(gathered, token_type_ids, visual, seg_table, v_seg_table,
             g, b, vg, vb):
    batch, seq_l, d = gathered.shape
    f = visual.shape[1]
    bb = 8
    grid = (batch // bb,)
    return pl.pallas_call(
        _tc_body,
        grid=grid,
        in_specs=[
            pl.BlockSpec((bb, seq_l, d), lambda i: (i, 0, 0)),
            pl.BlockSpec((bb, seq_l, 1), lambda i: (i, 0, 0)),
            pl.BlockSpec((bb, f, d), lambda i: (i, 0, 0)),
            pl.BlockSpec((2, d), lambda i: (0, 0)),
            pl.BlockSpec((1, d), lambda i: (0, 0)),
            pl.BlockSpec((1, d), lambda i: (0, 0)),
            pl.BlockSpec((1, d), lambda i: (0, 0)),
            pl.BlockSpec((1, d), lambda i: (0, 0)),
            pl.BlockSpec((1, d), lambda i: (0, 0)),
        ],
        out_specs=pl.BlockSpec((bb, seq_l + f, d), lambda i: (i, 0, 0)),
        out_shape=jax.ShapeDtypeStruct((batch, seq_l + f, d), jnp.float32),
    )(gathered, token_type_ids, visual, seg_table, v_seg_table, g, b, vg, vb)


def kernel(input_ids, token_type_ids, input_mask, visual_embeds, visual_mask,
           tok_table, seg_table, v_seg_table, norm_gamma, norm_beta,
           vln_gamma, vln_beta):
    batch, seq_l = input_ids.shape
    d = tok_table.shape[1]
    ids = input_ids.astype(jnp.int32).reshape(-1)
    gathered = _sc_gather(tok_table, ids).reshape(batch, seq_l, d)
    emb = _tc_fuse(
        gathered,
        token_type_ids.astype(jnp.float32).reshape(batch, seq_l, 1),
        visual_embeds,
        seg_table,
        v_seg_table,
        norm_gamma.reshape(1, d),
        norm_beta.reshape(1, d),
        vln_gamma.reshape(1, d),
        vln_beta.reshape(1, d),
    )
    out_mask = jnp.concatenate([input_mask, visual_mask], axis=1)
    return (emb, out_mask)


# speedup vs baseline: 6.2213x; 1.2256x over previous
import functools

import jax
import jax.numpy as jnp
from jax import lax
from jax.experimental import pallas as pl
from jax.experimental.pallas import tpu as pltpu
from jax.experimental.pallas import tpu_sc as plsc

_EPS = 1e-12

_NC = 2
_NS = 16
_NW = _NC * _NS
_CHUNK = 128


def _sc_gather(table, ids_flat):
    n = ids_flat.shape[0]
    d = table.shape[1]
    n_chunks = n // (_NW * _CHUNK)
    idx = ids_flat.reshape(_NW, n_chunks, _CHUNK)
    mesh = plsc.VectorSubcoreMesh(core_axis_name="c", subcore_axis_name="s")

    @functools.partial(
        pl.kernel,
        mesh=mesh,
        out_type=jax.ShapeDtypeStruct((_NW, n_chunks, _CHUNK, d), jnp.float32),
        scratch_types=[
            pltpu.VMEM((n_chunks, _CHUNK), jnp.int32),
            pltpu.VMEM((_CHUNK, d), jnp.float32),
            pltpu.SemaphoreType.DMA,
        ],
    )
    def k(table_hbm, idx_hbm, out_hbm, idx_v, rows_v, sem):
        wid = lax.axis_index("s") * _NC + lax.axis_index("c")
        pltpu.sync_copy(idx_hbm.at[wid], idx_v)

        def body(c, carry):
            pltpu.async_copy(table_hbm.at[idx_v.at[c]], rows_v, sem).wait()
            pltpu.sync_copy(rows_v, out_hbm.at[wid, c])
            return carry

        lax.fori_loop(0, n_chunks, body, 0)

    return k(table, idx).reshape(n, d)


def _ln(x, g, b):
    mean = jnp.mean(x, axis=-1, keepdims=True)
    xc = x - mean
    var = jnp.mean(xc * xc, axis=-1, keepdims=True)
    return xc * lax.rsqrt(var + _EPS) * g + b


def _tc_body(tg_ref, tt_ref, vis_ref, seg_ref, vseg_ref, g_ref, b_ref,
             vg_ref, vb_ref, out_ref):
    seq_l = tg_ref.shape[1]
    ttf = tt_ref[...]
    seg0 = seg_ref[0, :]
    dseg = seg_ref[1, :] - seg0
    text = tg_ref[...] + seg0 + ttf * dseg
    g = g_ref[...]
    b = b_ref[...]
    out_ref[:, :seq_l, :] = _ln(text, g, b)
    v = _ln(vis_ref[...], vg_ref[...], vb_ref[...]) + vseg_ref[0, :]
    out_ref[:, seq_l:, :] = _ln(v, g, b)


def _tc_fuse(gathered, token_type_ids, visual, seg_table, v_seg_table,
             g, b, vg, vb):
    batch, seq_l, d = gathered.shape
    f = visual.shape[1]
    bb = 32
    grid = (batch // bb,)
    return pl.pallas_call(
        _tc_body,
        grid=grid,
        in_specs=[
            pl.BlockSpec((bb, seq_l, d), lambda i: (i, 0, 0)),
            pl.BlockSpec((bb, seq_l, 1), lambda i: (i, 0, 0)),
            pl.BlockSpec((bb, f, d), lambda i: (i, 0, 0)),
            pl.BlockSpec((2, d), lambda i: (0, 0)),
            pl.BlockSpec((1, d), lambda i: (0, 0)),
            pl.BlockSpec((1, d), lambda i: (0, 0)),
            pl.BlockSpec((1, d), lambda i: (0, 0)),
            pl.BlockSpec((1, d), lambda i: (0, 0)),
            pl.BlockSpec((1, d), lambda i: (0, 0)),
        ],
        out_specs=pl.BlockSpec((bb, seq_l + f, d), lambda i: (i, 0, 0)),
        out_shape=jax.ShapeDtypeStruct((batch, seq_l + f, d), jnp.float32),
    )(gathered, token_type_ids, visual, seg_table, v_seg_table, g, b, vg, vb)


def kernel(input_ids, token_type_ids, input_mask, visual_embeds, visual_mask,
           tok_table, seg_table, v_seg_table, norm_gamma, norm_beta,
           vln_gamma, vln_beta):
    batch, seq_l = input_ids.shape
    d = tok_table.shape[1]
    ids = input_ids.astype(jnp.int32).reshape(-1)
    gathered = _sc_gather(tok_table, ids).reshape(batch, seq_l, d)
    emb = _tc_fuse(
        gathered,
        token_type_ids.astype(jnp.float32).reshape(batch, seq_l, 1),
        visual_embeds,
        seg_table,
        v_seg_table,
        norm_gamma.reshape(1, d),
        norm_beta.reshape(1, d),
        vln_gamma.reshape(1, d),
        vln_beta.reshape(1, d),
    )
    out_mask = jnp.concatenate([input_mask, visual_mask], axis=1)
    return (emb, out_mask)


# speedup vs baseline: 6.4002x; 1.0288x over previous
import functools

import jax
import jax.numpy as jnp
from jax import lax
from jax.experimental import pallas as pl
from jax.experimental.pallas import tpu as pltpu
from jax.experimental.pallas import tpu_sc as plsc

_EPS = 1e-12

_NC = 2
_NS = 16
_NW = _NC * _NS
_CHUNK = 128


def _sc_gather(table, ids_flat):
    n = ids_flat.shape[0]
    d = table.shape[1]
    n_chunks = n // (_NW * _CHUNK)
    idx = ids_flat.reshape(_NW, n_chunks, _CHUNK)
    mesh = plsc.VectorSubcoreMesh(core_axis_name="c", subcore_axis_name="s")

    @functools.partial(
        pl.kernel,
        mesh=mesh,
        out_type=jax.ShapeDtypeStruct((_NW, n_chunks, _CHUNK, d), jnp.float32),
        scratch_types=[
            pltpu.VMEM((n_chunks, _CHUNK), jnp.int32),
            pltpu.VMEM((_CHUNK, d), jnp.float32),
            pltpu.SemaphoreType.DMA,
        ],
    )
    def k(table_hbm, idx_hbm, out_hbm, idx_v, rows_v, sem):
        wid = lax.axis_index("s") * _NC + lax.axis_index("c")
        pltpu.sync_copy(idx_hbm.at[wid], idx_v)

        def body(c, carry):
            pltpu.async_copy(table_hbm.at[idx_v.at[c]], rows_v, sem).wait()
            pltpu.sync_copy(rows_v, out_hbm.at[wid, c])
            return carry

        lax.fori_loop(0, n_chunks, body, 0)

    return k(table, idx).reshape(n, d)


def _ln(x, g, b):
    mean = jnp.mean(x, axis=-1, keepdims=True)
    xc = x - mean
    var = jnp.mean(xc * xc, axis=-1, keepdims=True)
    return xc * lax.rsqrt(var + _EPS) * g + b


def _tc_body(tg_ref, tt_ref, vis_ref, seg_ref, vseg_ref, g_ref, b_ref,
             vg_ref, vb_ref, out_ref):
    seq_l = tg_ref.shape[1]
    ttf = tt_ref[...]
    seg0 = seg_ref[0, :]
    dseg = seg_ref[1, :] - seg0
    text = tg_ref[...] + seg0 + ttf * dseg
    g = g_ref[...]
    b = b_ref[...]
    out_ref[:, :seq_l, :] = _ln(text, g, b)
    v = _ln(vis_ref[...], vg_ref[...], vb_ref[...]) + vseg_ref[0, :]
    out_ref[:, seq_l:, :] = _ln(v, g, b)


def _tc_fuse(gathered, token_type_ids, visual, seg_table, v_seg_table,
             g, b, vg, vb):
    batch, seq_l, d = gathered.shape
    f = visual.shape[1]
    bb = 64
    grid = (batch // bb,)
    return pl.pallas_call(
        _tc_body,
        grid=grid,
        in_specs=[
            pl.BlockSpec((bb, seq_l, d), lambda i: (i, 0, 0)),
            pl.BlockSpec((bb, seq_l, 1), lambda i: (i, 0, 0)),
            pl.BlockSpec((bb, f, d), lambda i: (i, 0, 0)),
            pl.BlockSpec((2, d), lambda i: (0, 0)),
            pl.BlockSpec((1, d), lambda i: (0, 0)),
            pl.BlockSpec((1, d), lambda i: (0, 0)),
            pl.BlockSpec((1, d), lambda i: (0, 0)),
            pl.BlockSpec((1, d), lambda i: (0, 0)),
            pl.BlockSpec((1, d), lambda i: (0, 0)),
        ],
        out_specs=pl.BlockSpec((bb, seq_l + f, d), lambda i: (i, 0, 0)),
        out_shape=jax.ShapeDtypeStruct((batch, seq_l + f, d), jnp.float32),
    )(gathered, token_type_ids, visual, seg_table, v_seg_table, g, b, vg, vb)


def kernel(input_ids, token_type_ids, input_mask, visual_embeds, visual_mask,
           tok_table, seg_table, v_seg_table, norm_gamma, norm_beta,
           vln_gamma, vln_beta):
    batch, seq_l = input_ids.shape
    d = tok_table.shape[1]
    ids = input_ids.astype(jnp.int32).reshape(-1)
    gathered = _sc_gather(tok_table, ids).reshape(batch, seq_l, d)
    emb = _tc_fuse(
        gathered,
        token_type_ids.astype(jnp.float32).reshape(batch, seq_l, 1),
        visual_embeds,
        seg_table,
        v_seg_table,
        norm_gamma.reshape(1, d),
        norm_beta.reshape(1, d),
        vln_gamma.reshape(1, d),
        vln_beta.reshape(1, d),
    )
    out_mask = jnp.concatenate([input_mask, visual_mask], axis=1)
    return (emb, out_mask)


# speedup vs baseline: 6.4169x; 1.0026x over previous
import functools

import jax
import jax.numpy as jnp
from jax import lax
from jax.experimental import pallas as pl
from jax.experimental.pallas import tpu as pltpu
from jax.experimental.pallas import tpu_sc as plsc

_EPS = 1e-12

_NC = 2
_NS = 16
_NW = _NC * _NS
_CHUNK = 128


def _pick_chunk(rows_per_worker):
    for c in (128, 104, 96, 80, 64, 40, 32, 16, 8):
        if rows_per_worker % c == 0:
            return c
    raise ValueError(rows_per_worker)


def _sc_gather(table, ids_flat):
    n = ids_flat.shape[0]
    d = table.shape[1]
    chunk = _pick_chunk(n // _NW)
    n_chunks = n // (_NW * chunk)
    idx = ids_flat.reshape(_NW, n_chunks, chunk)
    mesh = plsc.VectorSubcoreMesh(core_axis_name="c", subcore_axis_name="s")

    @functools.partial(
        pl.kernel,
        mesh=mesh,
        out_type=jax.ShapeDtypeStruct((_NW, n_chunks, chunk, d), jnp.float32),
        scratch_types=[
            pltpu.VMEM((n_chunks, chunk), jnp.int32),
            pltpu.VMEM((chunk, d), jnp.float32),
            pltpu.SemaphoreType.DMA,
        ],
    )
    def k(table_hbm, idx_hbm, out_hbm, idx_v, rows_v, sem):
        wid = lax.axis_index("s") * _NC + lax.axis_index("c")
        pltpu.sync_copy(idx_hbm.at[wid], idx_v)

        def body(c, carry):
            pltpu.async_copy(table_hbm.at[idx_v.at[c]], rows_v, sem).wait()
            pltpu.sync_copy(rows_v, out_hbm.at[wid, c])
            return carry

        lax.fori_loop(0, n_chunks, body, 0)

    return k(table, idx).reshape(n, d)


def _ln(x, g, b):
    mean = jnp.mean(x, axis=-1, keepdims=True)
    xc = x - mean
    var = jnp.mean(xc * xc, axis=-1, keepdims=True)
    return xc * lax.rsqrt(var + _EPS) * g + b


def _tc_body(tg_ref, tt_ref, vis_ref, seg_ref, vseg_ref, g_ref, b_ref,
             vg_ref, vb_ref, out_ref):
    seq_l = tg_ref.shape[1]
    ttf = tt_ref[...]
    seg0 = seg_ref[0, :]
    dseg = seg_ref[1, :] - seg0
    text = tg_ref[...] + seg0 + ttf * dseg
    g = g_ref[...]
    b = b_ref[...]
    out_ref[:, :seq_l, :] = _ln(text, g, b)
    v = _ln(vis_ref[...], vg_ref[...], vb_ref[...]) + vseg_ref[0, :]
    out_ref[:, seq_l:, :] = _ln(v, g, b)


def _tc_body_alias(full_ref, tg_ref, tt_ref, vis_ref, seg_ref, vseg_ref,
                   g_ref, b_ref, vg_ref, vb_ref, out_ref):
    _tc_body(tg_ref, tt_ref, vis_ref, seg_ref, vseg_ref, g_ref, b_ref,
             vg_ref, vb_ref, out_ref)


def _tc_fuse_slice(full, gathered, token_type_ids, visual, seg_table,
                   v_seg_table, g, b, vg, vb, slice_idx, batch):
    bs, seq_l, d = gathered.shape
    f = visual.shape[1]
    bb = min(64, bs)
    grid = (bs // bb,)
    base = slice_idx * (bs // bb)
    in_specs = [
        pl.BlockSpec((bb, seq_l, d), lambda i: (i, 0, 0)),
        pl.BlockSpec((bb, seq_l, 1), lambda i: (i, 0, 0)),
        pl.BlockSpec((bb, f, d), lambda i: (i, 0, 0)),
        pl.BlockSpec((2, d), lambda i: (0, 0)),
        pl.BlockSpec((1, d), lambda i: (0, 0)),
        pl.BlockSpec((1, d), lambda i: (0, 0)),
        pl.BlockSpec((1, d), lambda i: (0, 0)),
        pl.BlockSpec((1, d), lambda i: (0, 0)),
        pl.BlockSpec((1, d), lambda i: (0, 0)),
    ]
    out_spec = pl.BlockSpec((bb, seq_l + f, d), lambda i: (base + i, 0, 0))
    out_shape = jax.ShapeDtypeStruct((batch, seq_l + f, d), jnp.float32)
    args = (gathered, token_type_ids, visual, seg_table, v_seg_table,
            g, b, vg, vb)
    if full is None:
        return pl.pallas_call(
            _tc_body, grid=grid, in_specs=in_specs,
            out_specs=out_spec, out_shape=out_shape,
        )(*args)
    full_spec = pl.BlockSpec((8, 8, d), lambda i: (0, 0, 0))
    return pl.pallas_call(
        _tc_body_alias, grid=grid, in_specs=[full_spec] + in_specs,
        out_specs=out_spec, out_shape=out_shape,
        input_output_aliases={0: 0},
    )(full, *args)


def kernel(input_ids, token_type_ids, input_mask, visual_embeds, visual_mask,
           tok_table, seg_table, v_seg_table, norm_gamma, norm_beta,
           vln_gamma, vln_beta):
    batch, seq_l = input_ids.shape
    d = tok_table.shape[1]
    n_slices = 2
    bs = batch // n_slices
    ids = input_ids.astype(jnp.int32)
    ttf = token_type_ids.astype(jnp.float32).reshape(batch, seq_l, 1)
    g = norm_gamma.reshape(1, d)
    b = norm_beta.reshape(1, d)
    vg = vln_gamma.reshape(1, d)
    vb = vln_beta.reshape(1, d)
    gathered = [
        _sc_gather(tok_table, ids[s * bs:(s + 1) * bs].reshape(-1))
        .reshape(bs, seq_l, d)
        for s in range(n_slices)
    ]
    full = None
    for s in range(n_slices):
        full = _tc_fuse_slice(
            full, gathered[s],
            ttf[s * bs:(s + 1) * bs],
            visual_embeds[s * bs:(s + 1) * bs],
            seg_table, v_seg_table, g, b, vg, vb, s, batch,
        )
    out_mask = jnp.concatenate([input_mask, visual_mask], axis=1)
    return (full, out_mask)
